# baseline (speedup 1.0000x reference)
import jax
import jax.numpy as jnp
from jax import lax
from jax.experimental import pallas as pl
from jax.experimental.pallas import tpu as pltpu
from jax.experimental.pallas import tpu_sc as plsc

NY = 800
NX = 700
NXP = 768
NCH = 36
BATCH = 4
N_VOX = 160000
PBATCH = NY * NXP
TOTP = BATCH * PBATCH
HALF = TOTP // 2
OUT_OF_RANGE = 1 << 29

NC = 2
NS = 16
NW = NC * NS
W = HALF // NW
CHUNK = 3200
N_CHUNKS = N_VOX // CHUNK
ONE_LO = 0x3F80
ONE_HI = 0x3F800000


def _prep_body(vi_ref, c_ref, bs_ref, t_ref, i_ref):
    bs = bs_ref[0, 0]
    b = c_ref[0:1, :]
    y = c_ref[2:3, :]
    x = c_ref[3:4, :]
    t = (b * NY + y) * NXP + x
    t_ref[...] = jnp.where(b < bs, t, jnp.int32(OUT_OF_RANGE))[0]
    i_ref[...] = jnp.sum(vi_ref[...], axis=0) * (1.0 / 8.0)


def _prep(vox_iT, coordsT, bs_arr):
    blk = 16384
    grid = (pl.cdiv(N_VOX, blk),)
    return pl.pallas_call(
        _prep_body,
        grid=grid,
        in_specs=[
            pl.BlockSpec((8, blk), lambda i: (0, i)),
            pl.BlockSpec((4, blk), lambda i: (0, i)),
            pl.BlockSpec(memory_space=pltpu.SMEM),
        ],
        out_specs=[
            pl.BlockSpec((blk,), lambda i: (i,)),
            pl.BlockSpec((blk,), lambda i: (i,)),
        ],
        out_shape=[
            jax.ShapeDtypeStruct((N_VOX,), jnp.int32),
            jax.ShapeDtypeStruct((N_VOX,), jnp.float32),
        ],
    )(vox_iT, coordsT, bs_arr)


def _sc_scatter_body(t_hbm, i_hbm, occ_hbm, int_hbm, occ_v, int_v, t_buf, i_buf,
                     sem0, sem1):
    cid = lax.axis_index("c")
    sid = lax.axis_index("s")
    wid = sid * NC + cid
    wlo = wid * W
    zi = jnp.zeros((16,), jnp.int32)

    @plsc.parallel_loop(0, W // 16, unroll=8)
    def zero_occ(j):
        occ_v[pl.ds(j * 16, 16)] = zi

    one_lo = jnp.full((16,), ONE_LO, jnp.int32)
    one_hi = jnp.full((16,), ONE_HI, jnp.int32)
    wvec = jnp.full((16,), W, jnp.int32)
    zvec = jnp.zeros((16,), jnp.int32)
    sems = (sem0, sem1)

    def issue(c, slot, sem):
        pltpu.async_copy(t_hbm.at[pl.ds(c * CHUNK, CHUNK)], t_buf.at[slot], sem)
        pltpu.async_copy(i_hbm.at[pl.ds(c * CHUNK, CHUNK)], i_buf.at[slot], sem)

    def wait_slot(slot, sem):
        pltpu.make_async_copy(t_hbm.at[pl.ds(0, CHUNK)], t_buf.at[slot], sem).wait()
        pltpu.make_async_copy(i_hbm.at[pl.ds(0, CHUNK)], i_buf.at[slot], sem).wait()

    issue(0, 0, sem0)
    issue(1, 1, sem1)

    def pair_body(p, carry):
        for slot in (0, 1):
            c = 2 * p + slot
            wait_slot(slot, sems[slot])

            @plsc.parallel_loop(0, CHUNK // 16, unroll=8)
            def inner(j):
                tv = t_buf[slot, pl.ds(j * 16, 16)]
                iv = i_buf[slot, pl.ds(j * 16, 16)]
                hi = tv >= HALF
                w = jnp.where(hi, tv - HALF, tv)
                d = w - wlo
                m = d.astype(jnp.uint32) < jnp.uint32(W)
                idx_int = d + jnp.where(hi, wvec, zvec)
                plsc.store_scatter(int_v, [idx_int], iv, mask=m)
                val = jnp.where(hi, one_hi, one_lo)
                plsc.addupdate_scatter(occ_v, [d], val, mask=m)

            @pl.when(c + 2 < N_CHUNKS)
            def _():
                issue(c + 2, slot, sems[slot])

        return carry

    lax.fori_loop(0, N_CHUNKS // 2, pair_body, None)

    pltpu.sync_copy(occ_v, occ_hbm.at[pl.ds(wlo, W)])
    pltpu.sync_copy(int_v.at[pl.ds(0, W)], int_hbm.at[pl.ds(wlo, W)])
    pltpu.sync_copy(int_v.at[pl.ds(W, W)], int_hbm.at[pl.ds(HALF + wlo, W)])


_sc_scatter = pl.kernel(
    _sc_scatter_body,
    out_type=[
        jax.ShapeDtypeStruct((HALF,), jnp.int32),
        jax.ShapeDtypeStruct((TOTP,), jnp.float32),
    ],
    mesh=plsc.VectorSubcoreMesh(core_axis_name="c", subcore_axis_name="s"),
    scratch_types=[
        pltpu.VMEM((W,), jnp.int32),
        pltpu.VMEM((2 * W,), jnp.float32),
        pltpu.VMEM((2, CHUNK), jnp.int32),
        pltpu.VMEM((2, CHUNK), jnp.float32),
        pltpu.SemaphoreType.DMA,
        pltpu.SemaphoreType.DMA,
    ],
    compiler_params=pltpu.CompilerParams(
        needs_layout_passes=False, skip_device_barrier=True),
)


def _canvas_body(occ_ref, int_ref, out_ref):
    b = pl.program_id(0)
    rb = out_ref.shape[2]
    wv = occ_ref[...]
    low = lax.bitcast_convert_type(wv << 16, jnp.float32)
    high = lax.bitcast_convert_type(wv & jnp.int32(-65536), jnp.float32)
    occf = jnp.where(b < 2, low, high)
    msk = jnp.where(b < 2, wv << 16, wv & jnp.int32(-65536)) != 0
    ivb = lax.bitcast_convert_type(int_ref[...], jnp.int32)
    intf = lax.bitcast_convert_type(
        jnp.where(msk, ivb, jnp.int32(0)), jnp.float32)
    occr = occf.reshape(rb, NXP)[:, :NX]
    intr = intf.reshape(rb, NXP)[:, :NX]
    out_ref[:, 0:35, :, :] = jnp.broadcast_to(occr[None, None], (1, 35, rb, NX))
    out_ref[:, 35:36, :, :] = intr[None, None]


def _canvas(occ_w2, int_p2):
    rb = 200
    pr = rb * NXP // 128
    grid = (BATCH, NY // rb)
    return pl.pallas_call(
        _canvas_body,
        grid=grid,
        in_specs=[
            pl.BlockSpec((pr, 128), lambda b, p: ((b % 2) * (NY // rb) + p, 0)),
            pl.BlockSpec((pr, 128), lambda b, p: (b * (NY // rb) + p, 0)),
        ],
        out_specs=pl.BlockSpec((1, NCH, rb, NX), lambda b, p: (b, 0, p, 0)),
        out_shape=jax.ShapeDtypeStruct((BATCH, NCH, NY, NX), jnp.float32),
    )(occ_w2, int_p2)


def kernel(vox_feats, num_points, coords, batch_size):
    del num_points
    vox_iT = vox_feats[:, :, 4].T
    coordsT = coords.T
    bs_arr = jnp.asarray(batch_size, jnp.int32).reshape(1, 1)
    t, inten = _prep(vox_iT, coordsT, bs_arr)
    occ_w, int_pl = _sc_scatter(t, inten)
    return _canvas(occ_w.reshape(HALF // 128, 128), int_pl.reshape(TOTP // 128, 128))


# speedup vs baseline: 1.0504x; 1.0504x over previous
import jax
import jax.numpy as jnp
from jax import lax
from jax.experimental import pallas as pl
from jax.experimental.pallas import tpu as pltpu
from jax.experimental.pallas import tpu_sc as plsc

NY = 800
NX = 700
NXP = 768
NCH = 36
BATCH = 4
N_VOX = 160000
PBATCH = NY * NXP
TOTP = BATCH * PBATCH
OUT_OF_RANGE = 1 << 29

NC = 2
NS = 16
NW = NC * NS
R = TOTP // NW
CHUNK = 3200
N_CHUNKS = N_VOX // CHUNK


def _prep_body(vi_ref, c_ref, bs_ref, t_ref, i_ref):
    bs = bs_ref[0, 0]
    b = c_ref[0:1, :]
    y = c_ref[2:3, :]
    x = c_ref[3:4, :]
    t = (b * NY + y) * NXP + x
    t_ref[...] = jnp.where(b < bs, t, jnp.int32(OUT_OF_RANGE))[0]
    i_ref[...] = jnp.sum(vi_ref[...], axis=0) * (1.0 / 8.0)


def _prep(vox_iT, coordsT, bs_arr):
    blk = 16384
    grid = (pl.cdiv(N_VOX, blk),)
    return pl.pallas_call(
        _prep_body,
        grid=grid,
        in_specs=[
            pl.BlockSpec((8, blk), lambda i: (0, i)),
            pl.BlockSpec((4, blk), lambda i: (0, i)),
            pl.BlockSpec(memory_space=pltpu.SMEM),
        ],
        out_specs=[
            pl.BlockSpec((blk,), lambda i: (i,)),
            pl.BlockSpec((blk,), lambda i: (i,)),
        ],
        out_shape=[
            jax.ShapeDtypeStruct((N_VOX,), jnp.int32),
            jax.ShapeDtypeStruct((N_VOX,), jnp.float32),
        ],
    )(vox_iT, coordsT, bs_arr)


def _sc_scatter_body(t_hbm, i_hbm, int_hbm, int_v, t_buf, i_buf, sem0, sem1):
    cid = lax.axis_index("c")
    sid = lax.axis_index("s")
    wid = sid * NC + cid
    lo = wid * R
    zf = jnp.zeros((16,), jnp.float32)

    @plsc.parallel_loop(0, R // 16, unroll=8)
    def zero_int(j):
        int_v[pl.ds(j * 16, 16)] = zf

    one = jnp.full((16,), 1, jnp.int32)
    sems = (sem0, sem1)

    def issue(c, slot, sem):
        pltpu.async_copy(t_hbm.at[pl.ds(c * CHUNK, CHUNK)], t_buf.at[slot], sem)
        pltpu.async_copy(i_hbm.at[pl.ds(c * CHUNK, CHUNK)], i_buf.at[slot], sem)

    def wait_slot(slot, sem):
        pltpu.make_async_copy(t_hbm.at[pl.ds(0, CHUNK)], t_buf.at[slot], sem).wait()
        pltpu.make_async_copy(i_hbm.at[pl.ds(0, CHUNK)], i_buf.at[slot], sem).wait()

    issue(0, 0, sem0)
    issue(1, 1, sem1)

    def pair_body(p, carry):
        for slot in (0, 1):
            c = 2 * p + slot
            wait_slot(slot, sems[slot])

            @plsc.parallel_loop(0, CHUNK // 16, unroll=8)
            def inner(j):
                tv = t_buf[slot, pl.ds(j * 16, 16)]
                iv = i_buf[slot, pl.ds(j * 16, 16)]
                d = tv - lo
                m = d.astype(jnp.uint32) < jnp.uint32(R)
                tagged = lax.bitcast_convert_type(
                    lax.bitcast_convert_type(iv, jnp.int32) | one, jnp.float32)
                plsc.store_scatter(int_v, [d], tagged, mask=m)

            @pl.when(c + 2 < N_CHUNKS)
            def _():
                issue(c + 2, slot, sems[slot])

        return carry

    lax.fori_loop(0, N_CHUNKS // 2, pair_body, None)

    pltpu.sync_copy(int_v, int_hbm.at[pl.ds(lo, R)])


_sc_scatter = pl.kernel(
    _sc_scatter_body,
    out_type=jax.ShapeDtypeStruct((TOTP,), jnp.float32),
    mesh=plsc.VectorSubcoreMesh(core_axis_name="c", subcore_axis_name="s"),
    scratch_types=[
        pltpu.VMEM((R,), jnp.float32),
        pltpu.VMEM((2, CHUNK), jnp.int32),
        pltpu.VMEM((2, CHUNK), jnp.float32),
        pltpu.SemaphoreType.DMA,
        pltpu.SemaphoreType.DMA,
    ],
    compiler_params=pltpu.CompilerParams(needs_layout_passes=False),
)


def _canvas_body(int_ref, out_ref):
    rb = out_ref.shape[2]
    bits = lax.bitcast_convert_type(int_ref[...], jnp.int32)
    occf = jnp.where(bits != 0, jnp.float32(1.0), jnp.float32(0.0))
    intf = lax.bitcast_convert_type(bits & jnp.int32(-2), jnp.float32)
    occr = occf.reshape(rb, NXP)[:, :NX]
    intr = intf.reshape(rb, NXP)[:, :NX]
    out_ref[:, 0:35, :, :] = jnp.broadcast_to(occr[None, None], (1, 35, rb, NX))
    out_ref[:, 35:36, :, :] = intr[None, None]


def _canvas(int_p2):
    rb = 200
    pr = rb * NXP // 128
    nb = NY // rb
    grid = (BATCH, nb)
    return pl.pallas_call(
        _canvas_body,
        grid=grid,
        in_specs=[
            pl.BlockSpec((pr, 128), lambda b, p: (b * nb + p, 0)),
        ],
        out_specs=pl.BlockSpec((1, NCH, rb, NX), lambda b, p: (b, 0, p, 0)),
        out_shape=jax.ShapeDtypeStruct((BATCH, NCH, NY, NX), jnp.float32),
    )(int_p2)


def kernel(vox_feats, num_points, coords, batch_size):
    del num_points
    vox_iT = vox_feats[:, :, 4].T
    coordsT = coords.T
    bs_arr = jnp.asarray(batch_size, jnp.int32).reshape(1, 1)
    t, inten = _prep(vox_iT, coordsT, bs_arr)
    int_pl = _sc_scatter(t, inten)
    return _canvas(int_pl.reshape(TOTP // 128, 128))
